# baseline (speedup 1.0000x reference)
import functools

import jax
import jax.numpy as jnp
from jax import lax
from jax.experimental import pallas as pl
from jax.experimental.pallas import tpu as pltpu
from jax.experimental.pallas import tpu_sc as plsc

N = 10000
E = 320000
D = 128
BN_EPS = 1e-5

NC = 2
NS = 16
NPH = 2
NR = NPH * NC
CHUNK = 128
CH = 162
EPT = CHUNK * CH
EP = EPT * NS

RSZ = 2800
ACC_R = 2816
RPT = ACC_R // NS
WBR = 16

BLK = 400
GRID = N // BLK
SPLIT = RSZ // BLK


def _mesh():
    return plsc.VectorSubcoreMesh(
        core_axis_name="c", subcore_axis_name="s", num_cores=NC, num_subcores=NS
    )


@functools.partial(
    pl.kernel,
    out_type=jax.ShapeDtypeStruct((NR, ACC_R, D), jnp.float32),
    mesh=_mesh(),
    scratch_types=[
        pltpu.VMEM((CH, CHUNK), jnp.int32),
        pltpu.VMEM((CH, CHUNK), jnp.int32),
        pltpu.VMEM((CHUNK, D), jnp.float32),
        pltpu.VMEM((CHUNK, D), jnp.float32),
        pltpu.VMEM((WBR, D), jnp.float32),
        pltpu.VMEM((WBR, D), jnp.float32),
        pltpu.VMEM_SHARED((ACC_R, D), jnp.float32),
        pltpu.VMEM((16,), jnp.int32),
        pltpu.SemaphoreType.DMA,
        pltpu.SemaphoreType.DMA,
    ],
)
def _sc_agg(g_hbm, src_hbm, dstB_hbm, flag_hbm, out_hbm, sidx, didx, r0, r1,
            zb, wb, acc, flg, gs0, gs1):
    c = lax.axis_index("c")
    s = lax.axis_index("s")
    base = s * RPT

    zv = jnp.zeros((16,), jnp.float32)
    ov = jnp.ones((16,), jnp.float32)

    def fill_z(i, carry):
        for k in range(D // 16):
            zb[i, pl.ds(k * 16, 16)] = zv
        return carry

    lax.fori_loop(0, WBR, fill_z, 0)
    pltpu.sync_copy(src_hbm.at[s], sidx)
    pltpu.sync_copy(flag_hbm, flg)
    isdeg = flg[...][0] == 1

    @pl.when(isdeg)
    def _():
        def fill_o(i, carry):
            for k in range(D // 16):
                r0[i, pl.ds(k * 16, 16)] = ov
            return carry

        lax.fori_loop(0, CHUNK, fill_o, 0)

    @pl.when(jnp.logical_not(isdeg))
    def _():
        pltpu.async_copy(g_hbm.at[sidx.at[0]], r0, gs0)
        pltpu.async_copy(g_hbm.at[sidx.at[1]], r1, gs1)

    def pair(t, carry):
        j0 = t * 2
        j1 = j0 + 1
        jn0 = jnp.where(j0 + 2 < CH, j0 + 2, j0 + 2 - CH)
        jn1 = jnp.where(j1 + 2 < CH, j1 + 2, j1 + 2 - CH)
        pltpu.make_async_copy(g_hbm.at[sidx.at[j0]], r0, gs0).wait()
        pltpu.sync_copy(r0, acc.at[didx.at[j0]], add=True)
        pltpu.async_copy(g_hbm.at[sidx.at[jn0]], r0, gs0)
        pltpu.make_async_copy(g_hbm.at[sidx.at[j1]], r1, gs1).wait()
        pltpu.sync_copy(r1, acc.at[didx.at[j1]], add=True)
        pltpu.async_copy(g_hbm.at[sidx.at[jn1]], r1, gs1)
        return carry

    def pair_deg(t, carry):
        j0 = t * 2
        j1 = j0 + 1
        pltpu.sync_copy(r0, acc.at[didx.at[j0]], add=True)
        pltpu.sync_copy(r0, acc.at[didx.at[j1]], add=True)
        return carry

    for p in range(NPH):
        rng = 2 * p + c
        pltpu.sync_copy(dstB_hbm.at[rng].at[s], didx)
        for w in range(RPT // WBR):
            pltpu.sync_copy(zb, acc.at[pl.ds(base + w * WBR, WBR)])
        plsc.subcore_barrier()

        @pl.when(isdeg)
        def _():
            lax.fori_loop(0, CH // 2, pair_deg, 0)

        @pl.when(jnp.logical_not(isdeg))
        def _():
            lax.fori_loop(0, CH // 2, pair, 0)

        plsc.subcore_barrier()

        for w in range(RPT // WBR):
            pltpu.sync_copy(acc.at[pl.ds(base + w * WBR, WBR)], wb)
            pltpu.sync_copy(wb, out_hbm.at[rng].at[pl.ds(base + w * WBR, WBR)])
        plsc.subcore_barrier()

    @pl.when(jnp.logical_not(isdeg))
    def _():
        pltpu.make_async_copy(g_hbm.at[sidx.at[0]], r0, gs0).wait()
        pltpu.make_async_copy(g_hbm.at[sidx.at[1]], r1, gs1).wait()


def _remap_body(d_ref, *o_refs):
    v = d_ref[...]
    for r, o_ref in enumerate(o_refs):
        lo = r * RSZ
        local = v - lo
        m = (v >= lo) & (v < lo + RSZ)
        o_ref[...] = jnp.where(m, local, RSZ)


def _remap(dstR):
    spec = pl.BlockSpec((1, CH, CHUNK), lambda i: (i, 0, 0))
    out = jax.ShapeDtypeStruct((NS, CH, CHUNK), jnp.int32)
    return pl.pallas_call(
        _remap_body, grid=(NS,), in_specs=[spec],
        out_specs=(spec,) * NR, out_shape=(out,) * NR,
    )(dstR)


def _tc_step_body(p_ref, pd_ref, x_ref, w_ref, a_ref, c_ref, bl_ref, f_ref,
                  u_ref, o_ref):
    u = u_ref[...]
    f = f_ref[...]
    us = u_ref[0:1, 0:1]
    dcol = us * p_ref[0][:, 0:1] + (1.0 - us) * pd_ref[0][:, 0:1]
    dinv = lax.rsqrt(dcol)
    sd = p_ref[0] * dinv
    t = u * x_ref[...] + (1.0 - u) * jnp.maximum(sd * a_ref[...] + c_ref[...], 0.0)
    g = jnp.dot(t, w_ref[...], preferred_element_type=jnp.float32,
                precision=lax.Precision.HIGHEST) * dinv
    o_ref[...] = f * (sd + bl_ref[...]) + (1.0 - f) * g


_ROW = pl.BlockSpec((BLK, D), lambda i: (i, 0))
_PART = pl.BlockSpec((1, BLK, D), lambda i: (i // SPLIT, i - SPLIT * (i // SPLIT), 0))
_PDEG = pl.BlockSpec((1, BLK, 16), lambda i: (i // SPLIT, i - SPLIT * (i // SPLIT), 0))
_WMAT = pl.BlockSpec((D, D), lambda i: (0, 0))
_VEC = pl.BlockSpec((1, D), lambda i: (0, 0))
_OUT = jax.ShapeDtypeStruct((N, D), jnp.float32)


def _tc_step(p, pdeg, x, w, a, c, bl, f, u):
    return pl.pallas_call(
        _tc_step_body, grid=(GRID,),
        in_specs=[_PART, _PDEG, _ROW, _WMAT, _VEC, _VEC, _VEC, _VEC, _VEC],
        out_specs=_ROW, out_shape=_OUT,
    )(p, pdeg, x, w, a, c, bl, f, u)


@jax.jit
def kernel(x, edge_index, W1, b1, g1, be1, W2, b2, g2, be2, W3, b3):
    loops = jnp.arange(N, dtype=jnp.int32)
    pad_src = jnp.zeros((EP - E - N,), jnp.int32)
    pad_dst = jnp.full((EP - E - N,), N, jnp.int32)
    srcR = jnp.concatenate([edge_index[0], loops, pad_src]).reshape(NS, CH, CHUNK)
    dstR = jnp.concatenate([edge_index[1], loops, pad_dst]).reshape(NS, CH, CHUNK)

    dstB = jnp.stack(_remap(dstR))

    sbn = 1.0 / jnp.sqrt(1.0 + BN_EPS)
    a1 = (g1 * sbn).reshape(1, D)
    c1 = (b1 * g1 * sbn + be1).reshape(1, D)
    a2 = (g2 * sbn).reshape(1, D)
    c2 = (b2 * g2 * sbn + be2).reshape(1, D)

    ones = jnp.ones((1, D), jnp.float32)
    zero = jnp.zeros((1, D), jnp.float32)
    eye = jnp.eye(D, dtype=jnp.float32)
    xs = (
        jnp.stack([W1, W2, W3, eye]),
        jnp.stack([ones, a1, a2, ones]),
        jnp.stack([zero, c1, c2, zero]),
        jnp.stack([zero, zero, zero, b3.reshape(1, D)]),
        jnp.stack([zero, zero, zero, ones]),
        jnp.stack([ones, zero, zero, zero]),
    )

    def body(i, carry):
        g, pdeg = carry
        W, a, c, bl, f, u = (lax.dynamic_index_in_dim(z, i, 0, keepdims=False)
                             for z in xs)
        f16 = jnp.broadcast_to((i == 0).astype(jnp.int32), (16,))
        p = _sc_agg(g, srcR, dstB, f16)
        g_next = _tc_step(p, pdeg, x, W, a, c, bl, f, u)
        pdeg_next = jnp.where(i == 0, p[:, :, :16], pdeg)
        return (g_next, pdeg_next)

    g0 = jnp.ones((N, D), jnp.float32)
    pdeg0 = jnp.ones((NR, ACC_R, 16), jnp.float32)
    n_steps = lax.optimization_barrier(jnp.int32(4))
    out, _ = lax.fori_loop(0, n_steps, body, (g0, pdeg0))
    return out


# speedup vs baseline: 1.2099x; 1.2099x over previous
import functools

import jax
import jax.numpy as jnp
from jax import lax
from jax.experimental import pallas as pl
from jax.experimental.pallas import tpu as pltpu
from jax.experimental.pallas import tpu_sc as plsc

N = 10000
E = 320000
D = 128
BN_EPS = 1e-5

NC = 2
NS = 16
NPH = 2
NR = NPH * NC
CHUNK = 120
CH = 172
EPT = CHUNK * CH
EP = EPT * NS

RSZ = 2800
ACC_R = 2816
RPT = ACC_R // NS
WBR = 16

BLK = 400
GRID = N // BLK
SPLIT = RSZ // BLK


def _mesh():
    return plsc.VectorSubcoreMesh(
        core_axis_name="c", subcore_axis_name="s", num_cores=NC, num_subcores=NS
    )


@functools.partial(
    pl.kernel,
    out_type=jax.ShapeDtypeStruct((NR, ACC_R, D), jnp.float32),
    mesh=_mesh(),
    scratch_types=[
        pltpu.VMEM((CH, CHUNK), jnp.int32),
        pltpu.VMEM((CH, CHUNK), jnp.int32),
        pltpu.VMEM((CHUNK, D), jnp.float32),
        pltpu.VMEM((CHUNK, D), jnp.float32),
        pltpu.VMEM((WBR, D), jnp.float32),
        pltpu.VMEM((WBR, D), jnp.float32),
        pltpu.VMEM_SHARED((ACC_R, D), jnp.float32),
        pltpu.VMEM((16,), jnp.int32),
        pltpu.SemaphoreType.DMA,
        pltpu.SemaphoreType.DMA,
    ],
)
def _sc_agg(g_hbm, src_hbm, dstB_hbm, flag_hbm, out_hbm, sidx, didx, r0, r1,
            zb, wb, acc, flg, gs0, gs1):
    c = lax.axis_index("c")
    s = lax.axis_index("s")
    base = s * RPT

    zv = jnp.zeros((16,), jnp.float32)
    ov = jnp.ones((16,), jnp.float32)

    def fill_z(i, carry):
        for k in range(D // 16):
            zb[i, pl.ds(k * 16, 16)] = zv
        return carry

    lax.fori_loop(0, WBR, fill_z, 0)
    pltpu.sync_copy(src_hbm.at[s], sidx)
    pltpu.sync_copy(flag_hbm, flg)
    isdeg = flg[...][0] == 1

    @pl.when(isdeg)
    def _():
        def fill_o(i, carry):
            for k in range(D // 16):
                r0[i, pl.ds(k * 16, 16)] = ov
            return carry

        lax.fori_loop(0, CHUNK, fill_o, 0)

    @pl.when(jnp.logical_not(isdeg))
    def _():
        pltpu.async_copy(g_hbm.at[sidx.at[0]], r0, gs0)
        pltpu.async_copy(g_hbm.at[sidx.at[1]], r1, gs1)

    def pair(t, carry):
        j0 = t * 2
        j1 = j0 + 1
        jn0 = jnp.where(j0 + 2 < CH, j0 + 2, j0 + 2 - CH)
        jn1 = jnp.where(j1 + 2 < CH, j1 + 2, j1 + 2 - CH)
        pltpu.make_async_copy(g_hbm.at[sidx.at[j0]], r0, gs0).wait()
        pltpu.sync_copy(r0, acc.at[didx.at[j0]], add=True)
        pltpu.async_copy(g_hbm.at[sidx.at[jn0]], r0, gs0)
        pltpu.make_async_copy(g_hbm.at[sidx.at[j1]], r1, gs1).wait()
        pltpu.sync_copy(r1, acc.at[didx.at[j1]], add=True)
        pltpu.async_copy(g_hbm.at[sidx.at[jn1]], r1, gs1)
        return carry

    def pair_deg(t, carry):
        j0 = t * 2
        j1 = j0 + 1
        pltpu.sync_copy(r0, acc.at[didx.at[j0]], add=True)
        pltpu.sync_copy(r0, acc.at[didx.at[j1]], add=True)
        return carry

    for p in range(NPH):
        rng = 2 * p + c
        pltpu.sync_copy(dstB_hbm.at[rng].at[s], didx)
        for w in range(RPT // WBR):
            pltpu.sync_copy(zb, acc.at[pl.ds(base + w * WBR, WBR)])
        plsc.subcore_barrier()

        @pl.when(isdeg)
        def _():
            lax.fori_loop(0, CH // 2, pair_deg, 0)

        @pl.when(jnp.logical_not(isdeg))
        def _():
            lax.fori_loop(0, CH // 2, pair, 0)

        plsc.subcore_barrier()

        for w in range(RPT // WBR):
            pltpu.sync_copy(acc.at[pl.ds(base + w * WBR, WBR)], wb)
            pltpu.sync_copy(wb, out_hbm.at[rng].at[pl.ds(base + w * WBR, WBR)])
        plsc.subcore_barrier()

    @pl.when(jnp.logical_not(isdeg))
    def _():
        pltpu.make_async_copy(g_hbm.at[sidx.at[0]], r0, gs0).wait()
        pltpu.make_async_copy(g_hbm.at[sidx.at[1]], r1, gs1).wait()


def _remap_body(d_ref, *o_refs):
    v = d_ref[...]
    for r, o_ref in enumerate(o_refs):
        lo = r * RSZ
        local = v - lo
        m = (v >= lo) & (v < lo + RSZ)
        o_ref[...] = jnp.where(m, local, RSZ)


def _remap(dstR):
    spec = pl.BlockSpec((1, CH, CHUNK), lambda i: (i, 0, 0))
    out = jax.ShapeDtypeStruct((NS, CH, CHUNK), jnp.int32)
    return pl.pallas_call(
        _remap_body, grid=(NS,), in_specs=[spec],
        out_specs=(spec,) * NR, out_shape=(out,) * NR,
    )(dstR)


def _tc_step_body(p_ref, pd_ref, x_ref, w_ref, a_ref, c_ref, bl_ref, f_ref,
                  u_ref, o_ref):
    u = u_ref[...]
    f = f_ref[...]
    us = u_ref[0:1, 0:1]
    dcol = us * p_ref[0][:, 0:1] + (1.0 - us) * pd_ref[0][:, 0:1]
    dinv = lax.rsqrt(dcol)
    sd = p_ref[0] * dinv
    t = u * x_ref[...] + (1.0 - u) * jnp.maximum(sd * a_ref[...] + c_ref[...], 0.0)
    g = jnp.dot(t, w_ref[...], preferred_element_type=jnp.float32,
                precision=lax.Precision.HIGHEST) * dinv
    o_ref[...] = f * (sd + bl_ref[...]) + (1.0 - f) * g


_ROW = pl.BlockSpec((BLK, D), lambda i: (i, 0))
_PART = pl.BlockSpec((1, BLK, D), lambda i: (i // SPLIT, i - SPLIT * (i // SPLIT), 0))
_PDEG = pl.BlockSpec((1, BLK, 16), lambda i: (i // SPLIT, i - SPLIT * (i // SPLIT), 0))
_WMAT = pl.BlockSpec((D, D), lambda i: (0, 0))
_VEC = pl.BlockSpec((1, D), lambda i: (0, 0))
_OUT = jax.ShapeDtypeStruct((N, D), jnp.float32)


def _tc_step(p, pdeg, x, w, a, c, bl, f, u):
    return pl.pallas_call(
        _tc_step_body, grid=(GRID,),
        in_specs=[_PART, _PDEG, _ROW, _WMAT, _VEC, _VEC, _VEC, _VEC, _VEC],
        out_specs=_ROW, out_shape=_OUT,
    )(p, pdeg, x, w, a, c, bl, f, u)


@jax.jit
def kernel(x, edge_index, W1, b1, g1, be1, W2, b2, g2, be2, W3, b3):
    loops = jnp.arange(N, dtype=jnp.int32)
    pad_src = jnp.zeros((EP - E - N,), jnp.int32)
    pad_dst = jnp.full((EP - E - N,), N, jnp.int32)
    srcR = jnp.concatenate([edge_index[0], loops, pad_src]).reshape(NS, CH, CHUNK)
    dstR = jnp.concatenate([edge_index[1], loops, pad_dst]).reshape(NS, CH, CHUNK)

    dstB = jnp.stack(_remap(dstR))

    sbn = 1.0 / jnp.sqrt(1.0 + BN_EPS)
    a1 = (g1 * sbn).reshape(1, D)
    c1 = (b1 * g1 * sbn + be1).reshape(1, D)
    a2 = (g2 * sbn).reshape(1, D)
    c2 = (b2 * g2 * sbn + be2).reshape(1, D)

    ones = jnp.ones((1, D), jnp.float32)
    zero = jnp.zeros((1, D), jnp.float32)
    eye = jnp.eye(D, dtype=jnp.float32)
    xs = (
        jnp.stack([W1, W2, W3, eye]),
        jnp.stack([ones, a1, a2, ones]),
        jnp.stack([zero, c1, c2, zero]),
        jnp.stack([zero, zero, zero, b3.reshape(1, D)]),
        jnp.stack([zero, zero, zero, ones]),
        jnp.stack([ones, zero, zero, zero]),
    )

    def body(i, carry):
        g, pdeg = carry
        W, a, c, bl, f, u = (lax.dynamic_index_in_dim(z, i, 0, keepdims=False)
                             for z in xs)
        f16 = jnp.broadcast_to((i == 0).astype(jnp.int32), (16,))
        p = _sc_agg(g, srcR, dstB, f16)
        g_next = _tc_step(p, pdeg, x, W, a, c, bl, f, u)
        pdeg_next = jnp.where(i == 0, p[:, :, :16], pdeg)
        return (g_next, pdeg_next)

    g0 = jnp.ones((N, D), jnp.float32)
    pdeg0 = jnp.ones((NR, ACC_R, 16), jnp.float32)
    n_steps = lax.optimization_barrier(jnp.int32(4))
    out, _ = lax.fori_loop(0, n_steps, body, (g0, pdeg0))
    return out
